# Initial kernel scaffold; baseline (speedup 1.0000x reference)
import jax
import jax.numpy as jnp
from jax.experimental import pallas as pl


def kernel(score, relevance, eth_label):
    raise NotImplementedError("write your pallas kernel here")


# speedup vs baseline: 11.4948x; 11.4948x over previous
import math

import jax
import jax.numpy as jnp
from jax.experimental import pallas as pl
from jax.experimental.pallas import tpu as pltpu

K = 10
NUM_MC = 25
M = 200
B = 1024
G = 4
BB = 16
LOG_K_FACT = float(math.factorial(K))


def _body(score_ref, g_ref, u_ref, rel_ref, eth_ref, out_ref):
    score = score_ref[...]
    mx = jnp.max(score, axis=-1, keepdims=True)
    e = jnp.exp(score - mx)
    ssum = jnp.sum(e, axis=-1, keepdims=True)
    s = e / ssum
    logits = jnp.log(s)
    total = jnp.sum(s, axis=-1)

    v = logits[:, None, :] + g_ref[...]
    s3 = jnp.broadcast_to(s[:, None, :], (BB, NUM_MC, M))
    rel3 = jnp.broadcast_to(rel_ref[...][:, None, :], (BB, NUM_MC, M))
    iota = jax.lax.broadcasted_iota(jnp.int32, (BB, NUM_MC, M), 2)

    eth3 = []
    for gch in range(G):
        eth3.append(jnp.broadcast_to(eth_ref[:, gch, :][:, None, :], (BB, NUM_MC, M)))

    ps = []
    relsum = jnp.zeros((BB, NUM_MC), jnp.float32)
    df = [jnp.zeros((BB, NUM_MC), jnp.float32) for _ in range(G)]
    dfeq = [jnp.zeros((BB, NUM_MC), jnp.float32) for _ in range(G)]

    for _ in range(K):
        m = jnp.max(v, axis=-1)
        eq = v == m[:, :, None]
        ii = jnp.min(jnp.where(eq, iota, M), axis=-1)
        imask = (iota == ii[:, :, None]).astype(jnp.float32)
        p_j = jnp.sum(imask * s3, axis=-1)
        r_j = jnp.sum(imask * rel3, axis=-1)
        ps.append(p_j)
        relsum = relsum + r_j
        for gch in range(G):
            e_j = jnp.sum(imask * eth3[gch], axis=-1)
            df[gch] = df[gch] + e_j * r_j
            dfeq[gch] = dfeq[gch] + e_j
        v = jnp.where(imask > 0, -jnp.inf, v)

    us = [u_ref[:, :, k] for k in range(K)]
    ranks = []
    for k in range(K):
        t = jnp.zeros((BB, NUM_MC), jnp.float32)
        for k2 in range(K):
            lt = (us[k2] < us[k]).astype(jnp.float32)
            if k2 < k:
                lt = lt + (us[k2] == us[k]).astype(jnp.float32)
            t = t + lt
        ranks.append(t)

    total3 = jnp.broadcast_to(total[:, None], (BB, NUM_MC))
    numer = ps[0]
    for k in range(1, K):
        numer = numer * ps[k]
    prod_d = total3
    for t in range(1, K):
        tf = jnp.float32(t)
        s_t = jnp.zeros((BB, NUM_MC), jnp.float32)
        for k in range(K):
            s_t = s_t + ps[k] * (ranks[k] < tf).astype(jnp.float32)
        prod_d = prod_d * (total3 - s_t)
    logp = jnp.log(LOG_K_FACT * numer / prod_d)

    zero = relsum == 0.0
    dfn = [jnp.where(zero, dfeq[gch] * (1.0 / K), df[gch] / relsum) for gch in range(G)]
    ssumd = dfn[0] + dfn[1] + dfn[2] + dfn[3]
    ent = jnp.zeros((BB, NUM_MC), jnp.float32)
    for gch in range(G):
        p = dfn[gch] / ssumd
        ent = ent - p * jnp.log(p)

    partial = jnp.sum(logp * ent, axis=(0, 1), keepdims=True) * (-1.0 / (NUM_MC * B))

    @pl.when(pl.program_id(0) == 0)
    def _():
        out_ref[...] = jnp.zeros((1, 1), jnp.float32)

    out_ref[...] += partial


def kernel(score, relevance, eth_label):
    key = jax.random.key(42)
    k1, k2 = jax.random.split(key)
    g = jax.random.gumbel(k1, (B, NUM_MC, M), dtype=jnp.float32)
    u = jax.random.uniform(k2, (B, NUM_MC, K))
    eth_t = jnp.transpose(eth_label, (0, 2, 1))

    out = pl.pallas_call(
        _body,
        grid=(B // BB,),
        in_specs=[
            pl.BlockSpec((BB, M), lambda i: (i, 0)),
            pl.BlockSpec((BB, NUM_MC, M), lambda i: (i, 0, 0)),
            pl.BlockSpec((BB, NUM_MC, K), lambda i: (i, 0, 0)),
            pl.BlockSpec((BB, M), lambda i: (i, 0)),
            pl.BlockSpec((BB, G, M), lambda i: (i, 0, 0)),
        ],
        out_specs=pl.BlockSpec((1, 1), lambda i: (0, 0)),
        out_shape=jax.ShapeDtypeStruct((1, 1), jnp.float32),
        compiler_params=pltpu.CompilerParams(
            dimension_semantics=("arbitrary",),
        ),
    )(score, g, u, relevance, eth_t)
    return out[0, 0]
